# Initial kernel scaffold; baseline (speedup 1.0000x reference)
import jax
import jax.numpy as jnp
from jax.experimental import pallas as pl


def kernel(features, conv_w, conv_b, cls_w, cls_b, bbox_w, bbox_b):
    raise NotImplementedError("write your pallas kernel here")


# speedup vs baseline: 3.7706x; 3.7706x over previous
import numpy as np
import jax
import jax.numpy as jnp
from jax.experimental import pallas as pl

_B = 2
_C = 256
_FEAT = 50
_STRIDE = 16
_IMG = 800.0
_A = 9
_PRE_NMS = 2000
_POST_NMS = 1000
_NMS_THRESH = 0.7
_MIN_SIZE = 1e-3
_NEG = -1e10
_CLIP = float(np.log(1000.0 / 16.0))

_PADH = _FEAT + 2
_ROWS_IN = 2712
_ROWS_OUT = 2600
_NTOP = 2048


def _make_anchors():
    sizes = np.array([128.0, 256.0, 512.0])
    ratios = np.array([0.5, 1.0, 2.0])
    ws, hs = [], []
    for s in sizes:
        for r in ratios:
            hs.append(s * np.sqrt(r))
            ws.append(s / np.sqrt(r))
    ws = np.array(ws)
    hs = np.array(hs)
    base = np.stack([-ws / 2.0, -hs / 2.0, ws / 2.0, hs / 2.0], axis=1)
    shifts = np.arange(_FEAT, dtype=np.float64) * _STRIDE
    sy, sx = np.meshgrid(shifts, shifts, indexing='ij')
    shift = np.stack([sx.ravel(), sy.ravel(), sx.ravel(), sy.ravel()], axis=1)
    anchors = (shift[:, None, :] + base[None, :, :]).reshape(-1, 4)
    return jnp.asarray(anchors, dtype=jnp.float32)


_ANCH = _make_anchors()


def _conv_kernel(x_ref, w_ref, b_ref, wh_ref, bh_ref, out_ref):
    acc = jnp.zeros((_ROWS_OUT, _C), jnp.float32)
    for k in range(9):
        d = (k // 3) * _PADH + (k % 3)
        xs = x_ref[pl.ds(d, _ROWS_OUT), :]
        wk = w_ref[pl.ds(k * _C, _C), :]
        acc = acc + jnp.dot(xs, wk, preferred_element_type=jnp.float32)
    t = jnp.maximum(acc + b_ref[0, :][None, :], 0.0)
    out_ref[:, :] = (jnp.dot(t, wh_ref[:, :], preferred_element_type=jnp.float32)
                     + bh_ref[0, :][None, :])


def _nms_kernel(s_ref, d_ref, a_ref, out_ref):
    s = s_ref[:, :]
    ax1 = a_ref[0:16, :]
    ay1 = a_ref[16:32, :]
    ax2 = a_ref[32:48, :]
    ay2 = a_ref[48:64, :]
    dx = d_ref[0:16, :]
    dy = d_ref[16:32, :]
    dw = d_ref[32:48, :]
    dh = d_ref[48:64, :]

    w = ax2 - ax1
    h = ay2 - ay1
    cx = ax1 + 0.5 * w
    cy = ay1 + 0.5 * h
    dw = jnp.minimum(dw, _CLIP)
    dh = jnp.minimum(dh, _CLIP)
    px = dx * w + cx
    py = dy * h + cy
    pw = jnp.exp(dw) * w
    ph = jnp.exp(dh) * h
    x1 = jnp.clip(px - 0.5 * pw, 0.0, _IMG)
    y1 = jnp.clip(py - 0.5 * ph, 0.0, _IMG)
    x2 = jnp.clip(px + 0.5 * pw, 0.0, _IMG)
    y2 = jnp.clip(py + 0.5 * ph, 0.0, _IMG)
    valid = ((x2 - x1) >= _MIN_SIZE) & ((y2 - y1) >= _MIN_SIZE)
    s = jnp.where(valid, s, _NEG)
    area = (x2 - x1) * (y2 - y1)

    ridx = (jax.lax.broadcasted_iota(jnp.int32, (16, 128), 0) * 128
            + jax.lax.broadcasted_iota(jnp.int32, (16, 128), 1))
    li = jax.lax.broadcasted_iota(jnp.int32, (1, 128), 1)

    def body(i, s):
        m = jnp.max(s)
        idx = jnp.min(jnp.where(s == m, ridx, jnp.int32(1 << 30)))
        oh = (ridx == idx).astype(jnp.float32)
        bx1 = jnp.sum(oh * x1)
        by1 = jnp.sum(oh * y1)
        bx2 = jnp.sum(oh * x2)
        by2 = jnp.sum(oh * y2)
        barea = jnp.sum(oh * area)
        iw = jnp.maximum(jnp.minimum(bx2, x2) - jnp.maximum(bx1, x1), 0.0)
        ih = jnp.maximum(jnp.minimum(by2, y2) - jnp.maximum(by1, y1), 0.0)
        inter = iw * ih
        iou = inter / (barea + area - inter + 1e-9)
        row = (jnp.where(li == 0, bx1, 0.0)
               + jnp.where(li == 1, by1, 0.0)
               + jnp.where(li == 2, bx2, 0.0)
               + jnp.where(li == 3, by2, 0.0)
               + jnp.where(li == 4, jax.nn.sigmoid(m), 0.0))
        out_ref[pl.ds(i, 1), :] = row
        return jnp.where(iou > _NMS_THRESH, jnp.float32(_NEG), s)

    jax.lax.fori_loop(0, _POST_NMS, body, s)


def kernel(features, conv_w, conv_b, cls_w, cls_b, bbox_w, bbox_b):
    x = jnp.transpose(features, (0, 2, 3, 1))
    x = jnp.pad(x, ((0, 0), (1, 1), (1, 1), (0, 0)))
    x = x.reshape(_B, _PADH * _PADH, _C)
    x = jnp.pad(x, ((0, 0), (0, _ROWS_IN - _PADH * _PADH), (0, 0)))
    x = x.reshape(_B * _ROWS_IN, _C)

    w9 = jnp.transpose(conv_w, (2, 3, 1, 0)).reshape(9 * _C, _C)
    wh = jnp.concatenate([cls_w.reshape(_A, _C).T,
                          bbox_w.reshape(4 * _A, _C).T], axis=1)
    wh = jnp.pad(wh, ((0, 0), (0, 128 - 5 * _A)))
    bh = jnp.pad(jnp.concatenate([cls_b, bbox_b]), (0, 128 - 5 * _A))
    bh = bh.reshape(1, 128)
    cb = conv_b.reshape(1, _C)

    heads = pl.pallas_call(
        _conv_kernel,
        grid=(_B,),
        in_specs=[
            pl.BlockSpec((_ROWS_IN, _C), lambda b: (b, 0)),
            pl.BlockSpec((9 * _C, _C), lambda b: (0, 0)),
            pl.BlockSpec((1, _C), lambda b: (0, 0)),
            pl.BlockSpec((_C, 128), lambda b: (0, 0)),
            pl.BlockSpec((1, 128), lambda b: (0, 0)),
        ],
        out_specs=pl.BlockSpec((_ROWS_OUT, 128), lambda b: (b, 0)),
        out_shape=jax.ShapeDtypeStruct((_B * _ROWS_OUT, 128), jnp.float32),
    )(x, w9, cb, wh, bh)

    hv = heads.reshape(_B, _FEAT, _PADH, 128)[:, :, :_FEAT, :]
    obj = hv[..., 0:_A].reshape(_B, _FEAT * _FEAT * _A)
    deltas = hv[..., _A:5 * _A].reshape(_B, _FEAT * _FEAT * _A, 4)

    top_s, top_idx = jax.lax.top_k(obj, _PRE_NMS)
    d_top = jnp.take_along_axis(deltas, top_idx[..., None], axis=1)
    a_top = _ANCH[top_idx]

    pad_n = _NTOP - _PRE_NMS
    s_p = jnp.pad(top_s, ((0, 0), (0, pad_n)), constant_values=_NEG)
    s_p = s_p.reshape(_B * 16, 128)
    d_p = jnp.pad(d_top, ((0, 0), (0, pad_n), (0, 0)))
    d_p = jnp.transpose(d_p, (0, 2, 1)).reshape(_B * 64, 128)
    a_p = jnp.pad(a_top, ((0, 0), (0, pad_n), (0, 0)))
    a_p = jnp.transpose(a_p, (0, 2, 1)).reshape(_B * 64, 128)

    out = pl.pallas_call(
        _nms_kernel,
        grid=(_B,),
        in_specs=[
            pl.BlockSpec((16, 128), lambda b: (b, 0)),
            pl.BlockSpec((64, 128), lambda b: (b, 0)),
            pl.BlockSpec((64, 128), lambda b: (b, 0)),
        ],
        out_specs=pl.BlockSpec((1024, 128), lambda b: (b, 0)),
        out_shape=jax.ShapeDtypeStruct((_B * 1024, 128), jnp.float32),
    )(s_p, d_p, a_p)

    return out.reshape(_B, 1024, 128)[:, :_POST_NMS, :5]


# speedup vs baseline: 7.1083x; 1.8852x over previous
import numpy as np
import jax
import jax.numpy as jnp
from jax.experimental import pallas as pl

_B = 2
_C = 256
_FEAT = 50
_STRIDE = 16
_IMG = 800.0
_A = 9
_PRE_NMS = 2000
_POST_NMS = 1000
_NMS_THRESH = 0.7
_MIN_SIZE = 1e-3
_NEG = -1e10
_CLIP = float(np.log(1000.0 / 16.0))

_PADH = _FEAT + 2
_ROWS_IN = 2712
_ROWS_OUT = 2600
_NTOP = 2048


def _make_anchors():
    sizes = np.array([128.0, 256.0, 512.0])
    ratios = np.array([0.5, 1.0, 2.0])
    ws, hs = [], []
    for s in sizes:
        for r in ratios:
            hs.append(s * np.sqrt(r))
            ws.append(s / np.sqrt(r))
    ws = np.array(ws)
    hs = np.array(hs)
    base = np.stack([-ws / 2.0, -hs / 2.0, ws / 2.0, hs / 2.0], axis=1)
    shifts = np.arange(_FEAT, dtype=np.float64) * _STRIDE
    sy, sx = np.meshgrid(shifts, shifts, indexing='ij')
    shift = np.stack([sx.ravel(), sy.ravel(), sx.ravel(), sy.ravel()], axis=1)
    anchors = (shift[:, None, :] + base[None, :, :]).reshape(-1, 4)
    return jnp.asarray(anchors, dtype=jnp.float32)


_ANCH = _make_anchors()


def _conv_kernel(x_ref, w_ref, b_ref, wh_ref, bh_ref, out_ref):
    acc = jnp.zeros((_ROWS_OUT, _C), jnp.float32)
    for k in range(9):
        d = (k // 3) * _PADH + (k % 3)
        xs = x_ref[pl.ds(d, _ROWS_OUT), :]
        wk = w_ref[pl.ds(k * _C, _C), :]
        acc = acc + jnp.dot(xs, wk, preferred_element_type=jnp.float32)
    t = jnp.maximum(acc + b_ref[0, :][None, :], 0.0)
    out_ref[:, :] = (jnp.dot(t, wh_ref[:, :], preferred_element_type=jnp.float32)
                     + bh_ref[0, :][None, :])


def _nms_kernel(s_ref, d_ref, a_ref, out_ref):
    s = s_ref[:, :].reshape(_B, 16, 128)
    a = a_ref[:, :].reshape(_B, 4, 16, 128)
    d = d_ref[:, :].reshape(_B, 4, 16, 128)
    ax1 = a[:, 0]
    ay1 = a[:, 1]
    ax2 = a[:, 2]
    ay2 = a[:, 3]
    dx = d[:, 0]
    dy = d[:, 1]
    dw = d[:, 2]
    dh = d[:, 3]

    w = ax2 - ax1
    h = ay2 - ay1
    cx = ax1 + 0.5 * w
    cy = ay1 + 0.5 * h
    dw = jnp.minimum(dw, _CLIP)
    dh = jnp.minimum(dh, _CLIP)
    px = dx * w + cx
    py = dy * h + cy
    pw = jnp.exp(dw) * w
    ph = jnp.exp(dh) * h
    x1 = jnp.clip(px - 0.5 * pw, 0.0, _IMG)
    y1 = jnp.clip(py - 0.5 * ph, 0.0, _IMG)
    x2 = jnp.clip(px + 0.5 * pw, 0.0, _IMG)
    y2 = jnp.clip(py + 0.5 * ph, 0.0, _IMG)
    valid = ((x2 - x1) >= _MIN_SIZE) & ((y2 - y1) >= _MIN_SIZE)
    s = jnp.where(valid, s, _NEG)
    area = (x2 - x1) * (y2 - y1)
    s5 = jnp.stack([x1, y1, x2, y2, area], axis=0)

    ridx = (jax.lax.broadcasted_iota(jnp.int32, (_B, 16, 128), 1) * 128
            + jax.lax.broadcasted_iota(jnp.int32, (_B, 16, 128), 2))
    li = jax.lax.broadcasted_iota(jnp.int32, (1, 1, 128), 2)

    def body(i, s):
        m = jnp.max(s, axis=(1, 2), keepdims=True)
        idx = jnp.min(jnp.where(s == m, ridx, jnp.int32(1 << 30)),
                      axis=(1, 2), keepdims=True)
        oh = ridx == idx
        red = jnp.sum(jnp.where(oh[None], s5, 0.0),
                      axis=(2, 3), keepdims=True)
        bx1, by1, bx2, by2, barea = (red[0], red[1], red[2], red[3], red[4])
        iw = jnp.maximum(jnp.minimum(bx2, x2) - jnp.maximum(bx1, x1), 0.0)
        ih = jnp.maximum(jnp.minimum(by2, y2) - jnp.maximum(by1, y1), 0.0)
        inter = iw * ih
        iou = inter / (barea + area - inter + 1e-9)
        row = (jnp.where(li == 0, bx1, 0.0)
               + jnp.where(li == 1, by1, 0.0)
               + jnp.where(li == 2, bx2, 0.0)
               + jnp.where(li == 3, by2, 0.0)
               + jnp.where(li == 4, jax.nn.sigmoid(m), 0.0))
        out_ref[pl.ds(i, 1), :] = row[0]
        out_ref[pl.ds(1024 + i, 1), :] = row[1]
        return jnp.where(iou > _NMS_THRESH, jnp.float32(_NEG), s)

    jax.lax.fori_loop(0, _POST_NMS, body, s)


def kernel(features, conv_w, conv_b, cls_w, cls_b, bbox_w, bbox_b):
    x = jnp.transpose(features, (0, 2, 3, 1))
    x = jnp.pad(x, ((0, 0), (1, 1), (1, 1), (0, 0)))
    x = x.reshape(_B, _PADH * _PADH, _C)
    x = jnp.pad(x, ((0, 0), (0, _ROWS_IN - _PADH * _PADH), (0, 0)))
    x = x.reshape(_B * _ROWS_IN, _C)

    w9 = jnp.transpose(conv_w, (2, 3, 1, 0)).reshape(9 * _C, _C)
    wh = jnp.concatenate([cls_w.reshape(_A, _C).T,
                          bbox_w.reshape(4 * _A, _C).T], axis=1)
    wh = jnp.pad(wh, ((0, 0), (0, 128 - 5 * _A)))
    bh = jnp.pad(jnp.concatenate([cls_b, bbox_b]), (0, 128 - 5 * _A))
    bh = bh.reshape(1, 128)
    cb = conv_b.reshape(1, _C)

    heads = pl.pallas_call(
        _conv_kernel,
        grid=(_B,),
        in_specs=[
            pl.BlockSpec((_ROWS_IN, _C), lambda b: (b, 0)),
            pl.BlockSpec((9 * _C, _C), lambda b: (0, 0)),
            pl.BlockSpec((1, _C), lambda b: (0, 0)),
            pl.BlockSpec((_C, 128), lambda b: (0, 0)),
            pl.BlockSpec((1, 128), lambda b: (0, 0)),
        ],
        out_specs=pl.BlockSpec((_ROWS_OUT, 128), lambda b: (b, 0)),
        out_shape=jax.ShapeDtypeStruct((_B * _ROWS_OUT, 128), jnp.float32),
    )(x, w9, cb, wh, bh)

    hv = heads.reshape(_B, _FEAT, _PADH, 128)[:, :, :_FEAT, :]
    obj = hv[..., 0:_A].reshape(_B, _FEAT * _FEAT * _A)
    deltas = hv[..., _A:5 * _A].reshape(_B, _FEAT * _FEAT * _A, 4)

    top_s, top_idx = jax.lax.top_k(obj, _PRE_NMS)
    d_top = jnp.take_along_axis(deltas, top_idx[..., None], axis=1)
    a_top = _ANCH[top_idx]

    pad_n = _NTOP - _PRE_NMS
    s_p = jnp.pad(top_s, ((0, 0), (0, pad_n)), constant_values=_NEG)
    s_p = s_p.reshape(_B * 16, 128)
    d_p = jnp.pad(d_top, ((0, 0), (0, pad_n), (0, 0)))
    d_p = jnp.transpose(d_p, (0, 2, 1)).reshape(_B * 64, 128)
    a_p = jnp.pad(a_top, ((0, 0), (0, pad_n), (0, 0)))
    a_p = jnp.transpose(a_p, (0, 2, 1)).reshape(_B * 64, 128)

    out = pl.pallas_call(
        _nms_kernel,
        grid=(1,),
        in_specs=[
            pl.BlockSpec((_B * 16, 128), lambda b: (0, 0)),
            pl.BlockSpec((_B * 64, 128), lambda b: (0, 0)),
            pl.BlockSpec((_B * 64, 128), lambda b: (0, 0)),
        ],
        out_specs=pl.BlockSpec((_B * 1024, 128), lambda b: (0, 0)),
        out_shape=jax.ShapeDtypeStruct((_B * 1024, 128), jnp.float32),
    )(s_p, d_p, a_p)

    return out.reshape(_B, 1024, 128)[:, :_POST_NMS, :5]
